# baseline (speedup 1.0000x reference)
import functools

import jax
import jax.numpy as jnp
from jax import lax
from jax.experimental import pallas as pl
from jax.experimental.pallas import tpu as pltpu
from jax.experimental.pallas import tpu_sc as plsc

_N = 10000
_E = 160000
_NC = 2
_NS = 16
_NPC = _N // _NC
_RPT = 320
_DUMMY = _RPT * _NS
_ACC_ROWS = _DUMMY + 8
_K = 64
_NBLK = 160
_EPT = _NBLK * _K
_EPAD = _EPT * _NS
_LAST = _NPC - _RPT * (_NS - 1)
_NBUF = 4
_DEPTH = 2


_EPW = _EPAD // (_NC * _NS)
_NBLK1 = _EPT // _K


def _mp1_body(G, *refs):
    hw_parts = refs[:G]
    (src_v2, dst_v2, ew_1d, out, src_v, idx_v, ew_v) = refs[G:G + 7]
    bufs = list(refs[G + 7:G + 7 + _NBUF])
    acc = refs[G + 7 + _NBUF]
    gsems = list(refs[G + 8 + _NBUF:G + 8 + 2 * _NBUF])
    ssems = list(refs[G + 8 + 2 * _NBUF:G + 8 + 3 * _NBUF])
    c = lax.axis_index("c")
    s = lax.axis_index("s")
    row0 = s * _RPT
    blk0 = s * _NBLK1

    pltpu.sync_copy(src_v2.at[pl.ds(blk0, _NBLK1)], src_v)
    pltpu.sync_copy(dst_v2.at[pl.ds(blk0, _NBLK1)], idx_v)
    pltpu.sync_copy(ew_1d.at[pl.ds(s * _EPT, _EPT)], ew_v.at[pl.ds(0, _EPT)])

    lo = c * _NPC

    def _mk_idx(j, carry):
        for t in range(_K // 16):
            v = idx_v[j, pl.ds(t * 16, 16)]
            loc = v - lo
            ok = (loc >= 0) & (loc < _NPC)
            idx_v[j, pl.ds(t * 16, 16)] = jnp.where(ok, loc, _DUMMY)
        return carry

    lax.fori_loop(0, _NBLK1, _mk_idx, 0)

    zero = jnp.zeros((16,), jnp.float32)

    for g in range(G):
        hw = hw_parts[g]

        def _zero_buf(e, carry):
            for d in range(8):
                bufs[0][e, pl.ds(d * 16, 16)] = zero
            return carry

        lax.fori_loop(0, _K, _zero_buf, 0)
        for q in range(_RPT // _K):
            pltpu.sync_copy(bufs[0], acc.at[pl.ds(row0 + q * _K, _K)])

        plsc.subcore_barrier()

        def _issue(j, b):
            pltpu.async_copy(hw.at[src_v.at[j]], bufs[b], gsems[b])

        def _gwait(b):
            pltpu.make_async_copy(hw.at[src_v.at[0]], bufs[b],
                                  gsems[b]).wait()

        def _scat(j, b):
            pltpu.async_copy(bufs[b], acc.at[idx_v.at[j]], ssems[b],
                             add=True)

        def _swait(b):
            pltpu.make_async_copy(bufs[b], acc.at[idx_v.at[0]],
                                  ssems[b]).wait()

        def _scale(j, b):
            buf = bufs[b]

            def _grp(q, carry):
                ews = ew_v[pl.ds(j * _K + q * 16, 16)]
                for i in range(16):
                    e = q * 16 + i
                    ew16 = lax.gather(
                        ews, jnp.full((16, 1), i, jnp.int32),
                        lax.GatherDimensionNumbers(
                            offset_dims=(), collapsed_slice_dims=(0,),
                            start_index_map=(0,)),
                        slice_sizes=(1,),
                        mode=lax.GatherScatterMode.PROMISE_IN_BOUNDS)
                    for d in range(8):
                        buf[e, pl.ds(d * 16, 16)] = (
                            buf[e, pl.ds(d * 16, 16)] * ew16)
                return carry

            lax.fori_loop(0, _K // 16, _grp, 0)

        for b in range(_DEPTH):
            _issue(b, b)

        def _octet(i, carry):
            for p in range(_NBUF):
                j = i * _NBUF + p
                t = (p + _DEPTH) % _NBUF
                _gwait(p)
                if p < _NBUF - _DEPTH:
                    @pl.when(i >= 1)
                    def _():
                        _swait(t)

                    _issue(j + _DEPTH, t)
                else:
                    _swait(t)

                    @pl.when(j + _DEPTH < _NBLK1)
                    def _():
                        _issue(j + _DEPTH, t)

                _scale(j, p)
                _scat(j, p)
            return carry

        lax.fori_loop(0, _NBLK1 // _NBUF, _octet, 0)
        for b in range(_NBUF - _DEPTH, _NBUF):
            _swait(b)

        plsc.subcore_barrier()

        @pl.when(s < _NS - 1)
        def _():
            pltpu.sync_copy(acc.at[pl.ds(row0, _RPT)],
                            out.at[pl.ds(lo + row0, _RPT),
                                   pl.ds(g * 128, 128)])

        @pl.when(s == _NS - 1)
        def _():
            pltpu.sync_copy(acc.at[pl.ds((_NS - 1) * _RPT, _LAST)],
                            out.at[pl.ds(lo + (_NS - 1) * _RPT, _LAST),
                                   pl.ds(g * 128, 128)])


@functools.lru_cache(maxsize=None)
def _make_mp1(G):
    mesh = plsc.VectorSubcoreMesh(core_axis_name="c", subcore_axis_name="s")
    return functools.partial(
        pl.kernel,
        mesh=mesh,
        out_type=jax.ShapeDtypeStruct((_N, 128 * G), jnp.float32),
        compiler_params=pltpu.CompilerParams(use_tc_tiling_on_sc=False),
        scratch_types=(
            [
                pltpu.VMEM((_NBLK1, _K), jnp.int32),
                pltpu.VMEM((_NBLK1, _K), jnp.int32),
                pltpu.VMEM((_EPT + 16,), jnp.float32),
            ]
            + [pltpu.VMEM((_K, 128), jnp.float32)] * _NBUF
            + [pltpu.VMEM_SHARED((_ACC_ROWS, 128), jnp.float32)]
            + [pltpu.SemaphoreType.DMA] * (2 * _NBUF)
        ),
    )(functools.partial(_mp1_body, G))


_NBLK2 = _EPW // _K
_RPT2 = 632
_ACC2 = _RPT2 * _NS
_LAST2 = _N - _RPT2 * (_NS - 1)


def _mp2_body(*refs):
    (hw, src_v2, dst_v2, ew_1d, out, src_v, dst_v, ew_v) = refs[:8]
    bufs = list(refs[8:8 + _NBUF])
    acc = refs[8 + _NBUF]
    gsems = list(refs[9 + _NBUF:9 + 2 * _NBUF])
    ssems = list(refs[9 + 2 * _NBUF:9 + 3 * _NBUF])
    c = lax.axis_index("c")
    s = lax.axis_index("s")
    w = c * _NS + s
    row0 = s * _RPT2
    blk0 = w * _NBLK2

    pltpu.sync_copy(src_v2.at[pl.ds(blk0, _NBLK2)], src_v)
    pltpu.sync_copy(dst_v2.at[pl.ds(blk0, _NBLK2)], dst_v)
    pltpu.sync_copy(ew_1d.at[pl.ds(w * _EPW, _EPW)], ew_v.at[pl.ds(0, _EPW)])

    zero = jnp.zeros((16,), jnp.float32)

    def _zero_buf(e, carry):
        for d in range(4):
            bufs[0][e, pl.ds(d * 16, 16)] = zero
        return carry

    lax.fori_loop(0, _K, _zero_buf, 0)
    for q in range(_RPT2 // _K):
        pltpu.sync_copy(bufs[0], acc.at[pl.ds(row0 + q * _K, _K)])
    rem = _RPT2 - (_RPT2 // _K) * _K
    pltpu.sync_copy(bufs[0].at[pl.ds(0, rem)],
                    acc.at[pl.ds(row0 + _RPT2 - rem, rem)])

    plsc.subcore_barrier()

    def _issue(j, b):
        pltpu.async_copy(hw.at[src_v.at[j]], bufs[b], gsems[b])

    def _gwait(b):
        pltpu.make_async_copy(hw.at[src_v.at[0]], bufs[b], gsems[b]).wait()

    def _scat(j, b):
        pltpu.async_copy(bufs[b], acc.at[dst_v.at[j]], ssems[b], add=True)

    def _swait(b):
        pltpu.make_async_copy(bufs[b], acc.at[dst_v.at[0]], ssems[b]).wait()

    def _scale(j, b):
        buf = bufs[b]

        def _grp(q, carry):
            ews = ew_v[pl.ds(j * _K + q * 16, 16)]
            for i in range(16):
                e = q * 16 + i
                ew16 = lax.gather(
                    ews, jnp.full((16, 1), i, jnp.int32),
                    lax.GatherDimensionNumbers(
                        offset_dims=(), collapsed_slice_dims=(0,),
                        start_index_map=(0,)),
                    slice_sizes=(1,),
                    mode=lax.GatherScatterMode.PROMISE_IN_BOUNDS)
                for d in range(4):
                    buf[e, pl.ds(d * 16, 16)] = (
                        buf[e, pl.ds(d * 16, 16)] * ew16)
            return carry

        lax.fori_loop(0, _K // 16, _grp, 0)

    for b in range(_DEPTH):
        _issue(b, b)

    def _octet(i, carry):
        for p in range(_NBUF):
            j = i * _NBUF + p
            t = (p + _DEPTH) % _NBUF
            _gwait(p)
            if p < _NBUF - _DEPTH:
                @pl.when(i >= 1)
                def _():
                    _swait(t)

                _issue(j + _DEPTH, t)
            else:
                _swait(t)

                @pl.when(j + _DEPTH < _NBLK2)
                def _():
                    _issue(j + _DEPTH, t)

            _scale(j, p)
            _scat(j, p)
        return carry

    lax.fori_loop(0, _NBLK2 // _NBUF, _octet, 0)
    for b in range(_NBUF - _DEPTH, _NBUF):
        _swait(b)

    plsc.subcore_barrier()

    @pl.when(s < _NS - 1)
    def _():
        pltpu.sync_copy(acc.at[pl.ds(row0, _RPT2)],
                        out.at[c, pl.ds(row0, _RPT2)])

    @pl.when(s == _NS - 1)
    def _():
        pltpu.sync_copy(acc.at[pl.ds((_NS - 1) * _RPT2, _LAST2)],
                        out.at[c, pl.ds((_NS - 1) * _RPT2, _LAST2)])


@functools.lru_cache(maxsize=None)
def _make_mp2():
    mesh = plsc.VectorSubcoreMesh(core_axis_name="c", subcore_axis_name="s")
    return functools.partial(
        pl.kernel,
        mesh=mesh,
        out_type=jax.ShapeDtypeStruct((_NC, _N, 64), jnp.float32),
        compiler_params=pltpu.CompilerParams(use_tc_tiling_on_sc=False),
        scratch_types=(
            [
                pltpu.VMEM((_NBLK2, _K), jnp.int32),
                pltpu.VMEM((_NBLK2, _K), jnp.int32),
                pltpu.VMEM((_EPW + 16,), jnp.float32),
            ]
            + [pltpu.VMEM((_K, 64), jnp.float32)] * _NBUF
            + [pltpu.VMEM_SHARED((_ACC2, 64), jnp.float32)]
            + [pltpu.SemaphoreType.DMA] * (2 * _NBUF)
        ),
    )(_mp2_body)


def _mm_block(odt, a_ref, b_ref, o_ref):
    o_ref[...] = jnp.dot(a_ref[...], b_ref[...],
                         preferred_element_type=jnp.float32).astype(odt)


def _mm(a, b, out_dtype=jnp.float32, bn=1000):
    n, k = a.shape
    h = b.shape[1]
    return pl.pallas_call(
        functools.partial(_mm_block, out_dtype),
        grid=(n // bn,),
        in_specs=[pl.BlockSpec((bn, k), lambda i: (i, 0)),
                  pl.BlockSpec((k, h), lambda i: (0, 0))],
        out_specs=pl.BlockSpec((bn, h), lambda i: (i, 0)),
        out_shape=jax.ShapeDtypeStruct((n, h), out_dtype),
    )(a, b)


def _mm2_block(a_ref, b_ref, o_ref):
    a = jnp.maximum(a_ref[...], 0.0)
    o_ref[...] = jnp.dot(a, b_ref[...], preferred_element_type=jnp.float32)


def _mm2(a, b, bn=1000):
    n, k = a.shape
    h = b.shape[1]
    return pl.pallas_call(
        _mm2_block,
        grid=(n // bn,),
        in_specs=[pl.BlockSpec((bn, k), lambda i: (i, 0)),
                  pl.BlockSpec((k, h), lambda i: (0, 0))],
        out_specs=pl.BlockSpec((bn, h), lambda i: (i, 0)),
        out_shape=jax.ShapeDtypeStruct((n, h), jnp.float32),
    )(a, b)


def _gram_block(zi_ref, zj_ref, o_ref):
    zi = zi_ref[0] + zi_ref[1]
    zj = zj_ref[0] + zj_ref[1]
    o_ref[...] = lax.dot_general(
        zi, zj, (((1,), (1,)), ((), ())),
        preferred_element_type=jnp.float32)


def _gram(z2, bz=200):
    _, n, h = z2.shape
    return pl.pallas_call(
        _gram_block,
        grid=(n // bz,),
        in_specs=[pl.BlockSpec((_NC, bz, h), lambda i: (0, i, 0)),
                  pl.BlockSpec((_NC, n, h), lambda i: (0, 0, 0))],
        out_specs=pl.BlockSpec((bz, n), lambda i: (i, 0)),
        out_shape=jax.ShapeDtypeStruct((n, n), jnp.float32),
    )(z2, z2)


def kernel(x, edge_index, edge_weight, W1, W2):
    pad = _EPAD - _E
    src2 = jnp.pad(edge_index[0], (0, pad)).reshape(_EPAD // _K, _K)
    dst2 = jnp.pad(edge_index[1], (0, pad)).reshape(_EPAD // _K, _K)
    ew1 = jnp.pad(edge_weight, (0, pad))

    hw1 = _mm(x, W1)
    agg1 = _make_mp1(2)(hw1[:, :128], hw1[:, 128:], src2, dst2, ew1)
    hw2 = _mm2(agg1, W2)
    z2 = _make_mp2()(hw2, src2, dst2, ew1)
    return _gram(z2).reshape(-1)


# speedup vs baseline: 1.0064x; 1.0064x over previous
import functools

import jax
import jax.numpy as jnp
from jax import lax
from jax.experimental import pallas as pl
from jax.experimental.pallas import tpu as pltpu
from jax.experimental.pallas import tpu_sc as plsc

_N = 10000
_E = 160000
_NC = 2
_NS = 16
_NPC = _N // _NC
_RPT = 320
_DUMMY = _RPT * _NS
_ACC_ROWS = _DUMMY + 8
_K = 64
_NBLK = 160
_EPT = _NBLK * _K
_EPAD = _EPT * _NS
_LAST = _NPC - _RPT * (_NS - 1)
_NBUF = 4
_DEPTH = 2


_EPW = _EPAD // (_NC * _NS)
_NBLK1 = _EPT // _K


def _mp1_body(G, *refs):
    hw_parts = refs[:G]
    (src_v2, dst_v2, ew_1d, out, src_v, idx_v, ew_v) = refs[G:G + 7]
    bufs = list(refs[G + 7:G + 7 + _NBUF])
    acc = refs[G + 7 + _NBUF]
    gsems = list(refs[G + 8 + _NBUF:G + 8 + 2 * _NBUF])
    ssems = list(refs[G + 8 + 2 * _NBUF:G + 8 + 3 * _NBUF])
    c = lax.axis_index("c")
    s = lax.axis_index("s")
    row0 = s * _RPT
    blk0 = s * _NBLK1

    pltpu.sync_copy(src_v2.at[pl.ds(blk0, _NBLK1)], src_v)
    pltpu.sync_copy(dst_v2.at[pl.ds(blk0, _NBLK1)], idx_v)
    pltpu.sync_copy(ew_1d.at[pl.ds(s * _EPT, _EPT)], ew_v.at[pl.ds(0, _EPT)])

    lo = c * _NPC

    def _mk_idx(j, carry):
        for t in range(_K // 16):
            v = idx_v[j, pl.ds(t * 16, 16)]
            loc = v - lo
            ok = (loc >= 0) & (loc < _NPC)
            idx_v[j, pl.ds(t * 16, 16)] = jnp.where(ok, loc, _DUMMY)
        return carry

    lax.fori_loop(0, _NBLK1, _mk_idx, 0)

    zero = jnp.zeros((16,), jnp.float32)

    for g in range(G):
        hw = hw_parts[g]

        def _zero_buf(e, carry):
            for d in range(8):
                bufs[0][e, pl.ds(d * 16, 16)] = zero
            return carry

        lax.fori_loop(0, _K, _zero_buf, 0)
        for q in range(_RPT // _K):
            pltpu.sync_copy(bufs[0], acc.at[pl.ds(row0 + q * _K, _K)])

        plsc.subcore_barrier()

        def _issue(j, b):
            pltpu.async_copy(hw.at[src_v.at[j]], bufs[b], gsems[b])

        def _gwait(b):
            pltpu.make_async_copy(hw.at[src_v.at[0]], bufs[b],
                                  gsems[b]).wait()

        def _scat(j, b):
            pltpu.async_copy(bufs[b], acc.at[idx_v.at[j]], ssems[b],
                             add=True)

        def _swait(b):
            pltpu.make_async_copy(bufs[b], acc.at[idx_v.at[0]],
                                  ssems[b]).wait()

        def _scale(j, b):
            buf = bufs[b]

            def _grp(q, carry):
                ews = ew_v[pl.ds(j * _K + q * 16, 16)]
                for i in range(16):
                    e = q * 16 + i
                    ew16 = lax.gather(
                        ews, jnp.full((16, 1), i, jnp.int32),
                        lax.GatherDimensionNumbers(
                            offset_dims=(), collapsed_slice_dims=(0,),
                            start_index_map=(0,)),
                        slice_sizes=(1,),
                        mode=lax.GatherScatterMode.PROMISE_IN_BOUNDS)
                    for d in range(8):
                        buf[e, pl.ds(d * 16, 16)] = (
                            buf[e, pl.ds(d * 16, 16)] * ew16)
                return carry

            lax.fori_loop(0, _K // 16, _grp, 0)

        for b in range(_DEPTH):
            _issue(b, b)

        def _octet(i, carry):
            for p in range(_NBUF):
                j = i * _NBUF + p
                t = (p + _DEPTH) % _NBUF
                _gwait(p)
                if p < _NBUF - _DEPTH:
                    @pl.when(i >= 1)
                    def _():
                        _swait(t)

                    _issue(j + _DEPTH, t)
                else:
                    _swait(t)

                    @pl.when(j + _DEPTH < _NBLK1)
                    def _():
                        _issue(j + _DEPTH, t)

                _scale(j, p)
                _scat(j, p)
            return carry

        lax.fori_loop(0, _NBLK1 // _NBUF, _octet, 0)
        for b in range(_NBUF - _DEPTH, _NBUF):
            _swait(b)

        plsc.subcore_barrier()

        @pl.when(s < _NS - 1)
        def _():
            pltpu.sync_copy(acc.at[pl.ds(row0, _RPT)],
                            out.at[pl.ds(lo + row0, _RPT),
                                   pl.ds(g * 128, 128)])

        @pl.when(s == _NS - 1)
        def _():
            pltpu.sync_copy(acc.at[pl.ds((_NS - 1) * _RPT, _LAST)],
                            out.at[pl.ds(lo + (_NS - 1) * _RPT, _LAST),
                                   pl.ds(g * 128, 128)])


@functools.lru_cache(maxsize=None)
def _make_mp1(G):
    mesh = plsc.VectorSubcoreMesh(core_axis_name="c", subcore_axis_name="s")
    return functools.partial(
        pl.kernel,
        mesh=mesh,
        out_type=jax.ShapeDtypeStruct((_N, 128 * G), jnp.float32),
        scratch_types=(
            [
                pltpu.VMEM((_NBLK1, _K), jnp.int32),
                pltpu.VMEM((_NBLK1, _K), jnp.int32),
                pltpu.VMEM((_EPT + 16,), jnp.float32),
            ]
            + [pltpu.VMEM((_K, 128), jnp.float32)] * _NBUF
            + [pltpu.VMEM_SHARED((_ACC_ROWS, 128), jnp.float32)]
            + [pltpu.SemaphoreType.DMA] * (2 * _NBUF)
        ),
    )(functools.partial(_mp1_body, G))


_NBLK2 = _EPW // _K
_RPT2 = 632
_ACC2 = _RPT2 * _NS
_LAST2 = _N - _RPT2 * (_NS - 1)


def _mp2_body(*refs):
    (hw, src_v2, dst_v2, ew_1d, out, src_v, dst_v, ew_v) = refs[:8]
    bufs = list(refs[8:8 + _NBUF])
    acc = refs[8 + _NBUF]
    gsems = list(refs[9 + _NBUF:9 + 2 * _NBUF])
    ssems = list(refs[9 + 2 * _NBUF:9 + 3 * _NBUF])
    c = lax.axis_index("c")
    s = lax.axis_index("s")
    w = c * _NS + s
    row0 = s * _RPT2
    blk0 = w * _NBLK2

    pltpu.sync_copy(src_v2.at[pl.ds(blk0, _NBLK2)], src_v)
    pltpu.sync_copy(dst_v2.at[pl.ds(blk0, _NBLK2)], dst_v)
    pltpu.sync_copy(ew_1d.at[pl.ds(w * _EPW, _EPW)], ew_v.at[pl.ds(0, _EPW)])

    zero = jnp.zeros((16,), jnp.float32)

    def _zero_buf(e, carry):
        for d in range(4):
            bufs[0][e, pl.ds(d * 16, 16)] = zero
        return carry

    lax.fori_loop(0, _K, _zero_buf, 0)
    for q in range(_RPT2 // _K):
        pltpu.sync_copy(bufs[0], acc.at[pl.ds(row0 + q * _K, _K)])
    rem = _RPT2 - (_RPT2 // _K) * _K
    pltpu.sync_copy(bufs[0].at[pl.ds(0, rem)],
                    acc.at[pl.ds(row0 + _RPT2 - rem, rem)])

    plsc.subcore_barrier()

    def _issue(j, b):
        pltpu.async_copy(hw.at[src_v.at[j]], bufs[b], gsems[b])

    def _gwait(b):
        pltpu.make_async_copy(hw.at[src_v.at[0]], bufs[b], gsems[b]).wait()

    def _scat(j, b):
        pltpu.async_copy(bufs[b], acc.at[dst_v.at[j]], ssems[b], add=True)

    def _swait(b):
        pltpu.make_async_copy(bufs[b], acc.at[dst_v.at[0]], ssems[b]).wait()

    def _scale(j, b):
        buf = bufs[b]

        def _grp(q, carry):
            ews = ew_v[pl.ds(j * _K + q * 16, 16)]
            for i in range(16):
                e = q * 16 + i
                ew16 = lax.gather(
                    ews, jnp.full((16, 1), i, jnp.int32),
                    lax.GatherDimensionNumbers(
                        offset_dims=(), collapsed_slice_dims=(0,),
                        start_index_map=(0,)),
                    slice_sizes=(1,),
                    mode=lax.GatherScatterMode.PROMISE_IN_BOUNDS)
                for d in range(4):
                    buf[e, pl.ds(d * 16, 16)] = (
                        buf[e, pl.ds(d * 16, 16)] * ew16)
            return carry

        lax.fori_loop(0, _K // 16, _grp, 0)

    for b in range(_DEPTH):
        _issue(b, b)

    def _octet(i, carry):
        for p in range(_NBUF):
            j = i * _NBUF + p
            t = (p + _DEPTH) % _NBUF
            _gwait(p)
            if p < _NBUF - _DEPTH:
                @pl.when(i >= 1)
                def _():
                    _swait(t)

                _issue(j + _DEPTH, t)
            else:
                _swait(t)

                @pl.when(j + _DEPTH < _NBLK2)
                def _():
                    _issue(j + _DEPTH, t)

            _scale(j, p)
            _scat(j, p)
        return carry

    lax.fori_loop(0, _NBLK2 // _NBUF, _octet, 0)
    for b in range(_NBUF - _DEPTH, _NBUF):
        _swait(b)

    plsc.subcore_barrier()

    @pl.when(s < _NS - 1)
    def _():
        pltpu.sync_copy(acc.at[pl.ds(row0, _RPT2)],
                        out.at[c, pl.ds(row0, _RPT2)])

    @pl.when(s == _NS - 1)
    def _():
        pltpu.sync_copy(acc.at[pl.ds((_NS - 1) * _RPT2, _LAST2)],
                        out.at[c, pl.ds((_NS - 1) * _RPT2, _LAST2)])


@functools.lru_cache(maxsize=None)
def _make_mp2():
    mesh = plsc.VectorSubcoreMesh(core_axis_name="c", subcore_axis_name="s")
    return functools.partial(
        pl.kernel,
        mesh=mesh,
        out_type=jax.ShapeDtypeStruct((_NC, _N, 64), jnp.float32),
        compiler_params=pltpu.CompilerParams(use_tc_tiling_on_sc=False),
        scratch_types=(
            [
                pltpu.VMEM((_NBLK2, _K), jnp.int32),
                pltpu.VMEM((_NBLK2, _K), jnp.int32),
                pltpu.VMEM((_EPW + 16,), jnp.float32),
            ]
            + [pltpu.VMEM((_K, 64), jnp.float32)] * _NBUF
            + [pltpu.VMEM_SHARED((_ACC2, 64), jnp.float32)]
            + [pltpu.SemaphoreType.DMA] * (2 * _NBUF)
        ),
    )(_mp2_body)


def _mm_block(odt, a_ref, b_ref, o_ref):
    o_ref[...] = jnp.dot(a_ref[...], b_ref[...],
                         preferred_element_type=jnp.float32).astype(odt)


def _mm(a, b, out_dtype=jnp.float32, bn=1000):
    n, k = a.shape
    h = b.shape[1]
    return pl.pallas_call(
        functools.partial(_mm_block, out_dtype),
        grid=(n // bn,),
        in_specs=[pl.BlockSpec((bn, k), lambda i: (i, 0)),
                  pl.BlockSpec((k, h), lambda i: (0, 0))],
        out_specs=pl.BlockSpec((bn, h), lambda i: (i, 0)),
        out_shape=jax.ShapeDtypeStruct((n, h), out_dtype),
    )(a, b)


def _mm2_block(a_ref, b_ref, o_ref):
    a = jnp.maximum(a_ref[...], 0.0)
    o_ref[...] = jnp.dot(a, b_ref[...], preferred_element_type=jnp.float32)


def _mm2(a, b, bn=1000):
    n, k = a.shape
    h = b.shape[1]
    return pl.pallas_call(
        _mm2_block,
        grid=(n // bn,),
        in_specs=[pl.BlockSpec((bn, k), lambda i: (i, 0)),
                  pl.BlockSpec((k, h), lambda i: (0, 0))],
        out_specs=pl.BlockSpec((bn, h), lambda i: (i, 0)),
        out_shape=jax.ShapeDtypeStruct((n, h), jnp.float32),
    )(a, b)


def _gram_block(zi_ref, zj_ref, o_ref):
    zi = zi_ref[0] + zi_ref[1]
    zj = zj_ref[0] + zj_ref[1]
    o_ref[...] = lax.dot_general(
        zi, zj, (((1,), (1,)), ((), ())),
        preferred_element_type=jnp.float32)


def _gram(z2, bz=200):
    _, n, h = z2.shape
    return pl.pallas_call(
        _gram_block,
        grid=(n // bz,),
        in_specs=[pl.BlockSpec((_NC, bz, h), lambda i: (0, i, 0)),
                  pl.BlockSpec((_NC, n, h), lambda i: (0, 0, 0))],
        out_specs=pl.BlockSpec((bz, n), lambda i: (i, 0)),
        out_shape=jax.ShapeDtypeStruct((n, n), jnp.float32),
    )(z2, z2)


def kernel(x, edge_index, edge_weight, W1, W2):
    pad = _EPAD - _E
    src2 = jnp.pad(edge_index[0], (0, pad)).reshape(_EPAD // _K, _K)
    dst2 = jnp.pad(edge_index[1], (0, pad)).reshape(_EPAD // _K, _K)
    ew1 = jnp.pad(edge_weight, (0, pad))

    hw1 = _mm(x, W1)
    agg1 = _make_mp1(2)(hw1[:, :128], hw1[:, 128:], src2, dst2, ew1)
    hw2 = _mm2(agg1, W2)
    z2 = _make_mp2()(hw2, src2, dst2, ew1)
    return _gram(z2).reshape(-1)


# speedup vs baseline: 1.0074x; 1.0009x over previous
import functools

import jax
import jax.numpy as jnp
from jax import lax
from jax.experimental import pallas as pl
from jax.experimental.pallas import tpu as pltpu
from jax.experimental.pallas import tpu_sc as plsc

_N = 10000
_E = 160000
_NC = 2
_NS = 16
_NPC = _N // _NC
_RPT = 320
_DUMMY = _RPT * _NS
_ACC_ROWS = _DUMMY + 8
_K = 64
_NBLK = 160
_EPT = _NBLK * _K
_EPAD = _EPT * _NS
_LAST = _NPC - _RPT * (_NS - 1)
_NBUF = 4
_DEPTH = 2


_EPW = _EPAD // (_NC * _NS)
_NBLK1 = _EPT // _K


def _mp1_body(G, *refs):
    hw_parts = refs[:G]
    (src_v2, dst_v2, ew_1d, out, src_v, idx_v, ew_v) = refs[G:G + 7]
    bufs = list(refs[G + 7:G + 7 + _NBUF])
    acc = refs[G + 7 + _NBUF]
    gsems = list(refs[G + 8 + _NBUF:G + 8 + 2 * _NBUF])
    ssems = list(refs[G + 8 + 2 * _NBUF:G + 8 + 3 * _NBUF])
    c = lax.axis_index("c")
    s = lax.axis_index("s")
    row0 = s * _RPT
    blk0 = s * _NBLK1

    pltpu.sync_copy(src_v2.at[pl.ds(blk0, _NBLK1)], src_v)
    pltpu.sync_copy(dst_v2.at[pl.ds(blk0, _NBLK1)], idx_v)
    pltpu.sync_copy(ew_1d.at[pl.ds(s * _EPT, _EPT)], ew_v.at[pl.ds(0, _EPT)])

    lo = c * _NPC

    def _mk_idx(j, carry):
        for t in range(_K // 16):
            v = idx_v[j, pl.ds(t * 16, 16)]
            loc = v - lo
            ok = (loc >= 0) & (loc < _NPC)
            idx_v[j, pl.ds(t * 16, 16)] = jnp.where(ok, loc, _DUMMY)
        return carry

    lax.fori_loop(0, _NBLK1, _mk_idx, 0)

    zero = jnp.zeros((16,), jnp.float32)

    for g in range(G):
        hw = hw_parts[g]

        def _zero_buf(e, carry):
            for d in range(8):
                bufs[0][e, pl.ds(d * 16, 16)] = zero
            return carry

        lax.fori_loop(0, _K, _zero_buf, 0)
        for q in range(_RPT // _K):
            pltpu.sync_copy(bufs[0], acc.at[pl.ds(row0 + q * _K, _K)])

        plsc.subcore_barrier()

        def _issue(j, b):
            pltpu.async_copy(hw.at[src_v.at[j]], bufs[b], gsems[b])

        def _gwait(b):
            pltpu.make_async_copy(hw.at[src_v.at[0]], bufs[b],
                                  gsems[b]).wait()

        def _scat(j, b):
            pltpu.async_copy(bufs[b], acc.at[idx_v.at[j]], ssems[b],
                             add=True)

        def _swait(b):
            pltpu.make_async_copy(bufs[b], acc.at[idx_v.at[0]],
                                  ssems[b]).wait()

        def _scale(j, b):
            buf = bufs[b]

            def _grp(q, carry):
                ews = ew_v[pl.ds(j * _K + q * 16, 16)]
                for i in range(16):
                    e = q * 16 + i
                    ew16 = lax.gather(
                        ews, jnp.full((16, 1), i, jnp.int32),
                        lax.GatherDimensionNumbers(
                            offset_dims=(), collapsed_slice_dims=(0,),
                            start_index_map=(0,)),
                        slice_sizes=(1,),
                        mode=lax.GatherScatterMode.PROMISE_IN_BOUNDS)
                    for d in range(8):
                        buf[e, pl.ds(d * 16, 16)] = (
                            buf[e, pl.ds(d * 16, 16)] * ew16)
                return carry

            lax.fori_loop(0, _K // 16, _grp, 0)

        for b in range(_DEPTH):
            _issue(b, b)

        def _octet(i, carry):
            for p in range(_NBUF):
                j = i * _NBUF + p
                t = (p + _DEPTH) % _NBUF
                _gwait(p)
                if p < _NBUF - _DEPTH:
                    @pl.when(i >= 1)
                    def _():
                        _swait(t)

                    _issue(j + _DEPTH, t)
                else:
                    _swait(t)

                    @pl.when(j + _DEPTH < _NBLK1)
                    def _():
                        _issue(j + _DEPTH, t)

                _scale(j, p)
                _scat(j, p)
            return carry

        lax.fori_loop(0, _NBLK1 // _NBUF, _octet, 0)
        for b in range(_NBUF - _DEPTH, _NBUF):
            _swait(b)

        plsc.subcore_barrier()

        @pl.when(s < _NS - 1)
        def _():
            pltpu.sync_copy(acc.at[pl.ds(row0, _RPT)],
                            out.at[pl.ds(lo + row0, _RPT),
                                   pl.ds(g * 128, 128)])

        @pl.when(s == _NS - 1)
        def _():
            pltpu.sync_copy(acc.at[pl.ds((_NS - 1) * _RPT, _LAST)],
                            out.at[pl.ds(lo + (_NS - 1) * _RPT, _LAST),
                                   pl.ds(g * 128, 128)])


@functools.lru_cache(maxsize=None)
def _make_mp1(G):
    mesh = plsc.VectorSubcoreMesh(core_axis_name="c", subcore_axis_name="s")
    return functools.partial(
        pl.kernel,
        mesh=mesh,
        out_type=jax.ShapeDtypeStruct((_N, 128 * G), jnp.float32),
        scratch_types=(
            [
                pltpu.VMEM((_NBLK1, _K), jnp.int32),
                pltpu.VMEM((_NBLK1, _K), jnp.int32),
                pltpu.VMEM((_EPT + 16,), jnp.float32),
            ]
            + [pltpu.VMEM((_K, 128), jnp.float32)] * _NBUF
            + [pltpu.VMEM_SHARED((_ACC_ROWS, 128), jnp.float32)]
            + [pltpu.SemaphoreType.DMA] * (2 * _NBUF)
        ),
    )(functools.partial(_mp1_body, G))


_NBLK2 = _EPW // _K
_RPT2 = 632
_ACC2 = _RPT2 * _NS
_LAST2 = _N - _RPT2 * (_NS - 1)


def _mp2_body(*refs):
    (hw, src_v2, dst_v2, ew_1d, out, src_v, dst_v, ew_v) = refs[:8]
    bufs = list(refs[8:8 + _NBUF])
    acc = refs[8 + _NBUF]
    gsems = list(refs[9 + _NBUF:9 + 2 * _NBUF])
    ssems = list(refs[9 + 2 * _NBUF:9 + 3 * _NBUF])
    c = lax.axis_index("c")
    s = lax.axis_index("s")
    w = c * _NS + s
    row0 = s * _RPT2
    blk0 = w * _NBLK2

    pltpu.sync_copy(src_v2.at[pl.ds(blk0, _NBLK2)], src_v)
    pltpu.sync_copy(dst_v2.at[pl.ds(blk0, _NBLK2)], dst_v)
    pltpu.sync_copy(ew_1d.at[pl.ds(w * _EPW, _EPW)], ew_v.at[pl.ds(0, _EPW)])

    zero = jnp.zeros((16,), jnp.float32)

    def _zero_buf(e, carry):
        for d in range(4):
            bufs[0][e, pl.ds(d * 16, 16)] = zero
        return carry

    lax.fori_loop(0, _K, _zero_buf, 0)
    for q in range(_RPT2 // _K):
        pltpu.sync_copy(bufs[0], acc.at[pl.ds(row0 + q * _K, _K)])
    rem = _RPT2 - (_RPT2 // _K) * _K
    pltpu.sync_copy(bufs[0].at[pl.ds(0, rem)],
                    acc.at[pl.ds(row0 + _RPT2 - rem, rem)])

    plsc.subcore_barrier()

    def _issue(j, b):
        pltpu.async_copy(hw.at[src_v.at[j]], bufs[b], gsems[b])

    def _gwait(b):
        pltpu.make_async_copy(hw.at[src_v.at[0]], bufs[b], gsems[b]).wait()

    def _scat(j, b):
        pltpu.async_copy(bufs[b], acc.at[dst_v.at[j]], ssems[b], add=True)

    def _swait(b):
        pltpu.make_async_copy(bufs[b], acc.at[dst_v.at[0]], ssems[b]).wait()

    def _scale(j, b):
        buf = bufs[b]

        def _grp(q, carry):
            ews = ew_v[pl.ds(j * _K + q * 16, 16)]
            for i in range(16):
                e = q * 16 + i
                ew16 = lax.gather(
                    ews, jnp.full((16, 1), i, jnp.int32),
                    lax.GatherDimensionNumbers(
                        offset_dims=(), collapsed_slice_dims=(0,),
                        start_index_map=(0,)),
                    slice_sizes=(1,),
                    mode=lax.GatherScatterMode.PROMISE_IN_BOUNDS)
                for d in range(4):
                    buf[e, pl.ds(d * 16, 16)] = (
                        buf[e, pl.ds(d * 16, 16)] * ew16)
            return carry

        lax.fori_loop(0, _K // 16, _grp, 0)

    for b in range(_DEPTH):
        _issue(b, b)

    def _octet(i, carry):
        for p in range(_NBUF):
            j = i * _NBUF + p
            t = (p + _DEPTH) % _NBUF
            _gwait(p)
            if p < _NBUF - _DEPTH:
                @pl.when(i >= 1)
                def _():
                    _swait(t)

                _issue(j + _DEPTH, t)
            else:
                _swait(t)

                @pl.when(j + _DEPTH < _NBLK2)
                def _():
                    _issue(j + _DEPTH, t)

            _scale(j, p)
            _scat(j, p)
        return carry

    lax.fori_loop(0, _NBLK2 // _NBUF, _octet, 0)
    for b in range(_NBUF - _DEPTH, _NBUF):
        _swait(b)

    plsc.subcore_barrier()

    @pl.when(s < _NS - 1)
    def _():
        pltpu.sync_copy(acc.at[pl.ds(row0, _RPT2)],
                        out.at[c, pl.ds(row0, _RPT2)])

    @pl.when(s == _NS - 1)
    def _():
        pltpu.sync_copy(acc.at[pl.ds((_NS - 1) * _RPT2, _LAST2)],
                        out.at[c, pl.ds((_NS - 1) * _RPT2, _LAST2)])


@functools.lru_cache(maxsize=None)
def _make_mp2():
    mesh = plsc.VectorSubcoreMesh(core_axis_name="c", subcore_axis_name="s")
    return functools.partial(
        pl.kernel,
        mesh=mesh,
        out_type=jax.ShapeDtypeStruct((_NC, _N, 64), jnp.float32),
        compiler_params=pltpu.CompilerParams(use_tc_tiling_on_sc=False),
        scratch_types=(
            [
                pltpu.VMEM((_NBLK2, _K), jnp.int32),
                pltpu.VMEM((_NBLK2, _K), jnp.int32),
                pltpu.VMEM((_EPW + 16,), jnp.float32),
            ]
            + [pltpu.VMEM((_K, 64), jnp.float32)] * _NBUF
            + [pltpu.VMEM_SHARED((_ACC2, 64), jnp.float32)]
            + [pltpu.SemaphoreType.DMA] * (2 * _NBUF)
        ),
    )(_mp2_body)


def _mm_block(odt, a_ref, b_ref, o_ref):
    o_ref[...] = jnp.dot(a_ref[...], b_ref[...],
                         preferred_element_type=jnp.float32).astype(odt)


def _mm(a, b, out_dtype=jnp.float32, bn=1000):
    n, k = a.shape
    h = b.shape[1]
    return pl.pallas_call(
        functools.partial(_mm_block, out_dtype),
        grid=(n // bn,),
        in_specs=[pl.BlockSpec((bn, k), lambda i: (i, 0)),
                  pl.BlockSpec((k, h), lambda i: (0, 0))],
        out_specs=pl.BlockSpec((bn, h), lambda i: (i, 0)),
        out_shape=jax.ShapeDtypeStruct((n, h), out_dtype),
    )(a, b)


def _mm2_block(a_ref, b_ref, o_ref):
    a = jnp.maximum(a_ref[...], 0.0)
    o_ref[...] = jnp.dot(a, b_ref[...], preferred_element_type=jnp.float32)


def _mm2(a, b, bn=1000):
    n, k = a.shape
    h = b.shape[1]
    return pl.pallas_call(
        _mm2_block,
        grid=(n // bn,),
        in_specs=[pl.BlockSpec((bn, k), lambda i: (i, 0)),
                  pl.BlockSpec((k, h), lambda i: (0, 0))],
        out_specs=pl.BlockSpec((bn, h), lambda i: (i, 0)),
        out_shape=jax.ShapeDtypeStruct((n, h), jnp.float32),
    )(a, b)


def _gram_block(zi_ref, zj_ref, o_ref):
    zi = zi_ref[0] + zi_ref[1]
    zj = zj_ref[0] + zj_ref[1]
    o_ref[...] = lax.dot_general(
        zi, zj, (((1,), (1,)), ((), ())),
        preferred_element_type=jnp.float32)


def _gram(z2, bz=400):
    _, n, h = z2.shape
    return pl.pallas_call(
        _gram_block,
        grid=(n // bz,),
        in_specs=[pl.BlockSpec((_NC, bz, h), lambda i: (0, i, 0)),
                  pl.BlockSpec((_NC, n, h), lambda i: (0, 0, 0))],
        out_specs=pl.BlockSpec((bz, n), lambda i: (i, 0)),
        out_shape=jax.ShapeDtypeStruct((n, n), jnp.float32),
    )(z2, z2)


def kernel(x, edge_index, edge_weight, W1, W2):
    pad = _EPAD - _E
    src2 = jnp.pad(edge_index[0], (0, pad)).reshape(_EPAD // _K, _K)
    dst2 = jnp.pad(edge_index[1], (0, pad)).reshape(_EPAD // _K, _K)
    ew1 = jnp.pad(edge_weight, (0, pad))

    hw1 = _mm(x, W1)
    agg1 = _make_mp1(2)(hw1[:, :128], hw1[:, 128:], src2, dst2, ew1)
    hw2 = _mm2(agg1, W2)
    z2 = _make_mp2()(hw2, src2, dst2, ew1)
    return _gram(z2).reshape(-1)
